# baseline (speedup 1.0000x reference)
import jax
import jax.numpy as jnp
from jax.experimental import pallas as pl
from jax.experimental.pallas import tpu as pltpu

N = 16384
IN_DIM = 512
H1 = 256
BLOCK = 4096
G = N // BLOCK


def _body(xa_ref, xb_ref, w1_ref, out_ref):
    w1 = w1_ref[...].astype(jnp.bfloat16)
    ha = jnp.dot(xa_ref[...].astype(jnp.bfloat16), w1,
                 preferred_element_type=jnp.float32)
    hb = jnp.dot(xb_ref[...].astype(jnp.bfloat16), w1,
                 preferred_element_type=jnp.float32)
    out_ref[...] = ha[:, :128] + ha[:, 128:] + hb[:, :128] + hb[:, 128:]


def kernel(x, W1, b1, W2, b2, W3, b3):
    return pl.pallas_call(
        _body,
        grid=(G,),
        in_specs=[
            pl.BlockSpec((BLOCK, IN_DIM), lambda i: (i, 0)),
            pl.BlockSpec((BLOCK, IN_DIM), lambda i: (G - 1 - i, 0)),
            pl.BlockSpec((IN_DIM, H1), lambda i: (0, 0)),
        ],
        out_specs=pl.BlockSpec((BLOCK, 128), lambda i: (i, 0)),
        out_shape=jax.ShapeDtypeStruct((N, 128), jnp.float32),
        compiler_params=pltpu.CompilerParams(
            dimension_semantics=("arbitrary",),
        ),
    )(x, x, W1)


# speedup vs baseline: 1.0306x; 1.0306x over previous
import jax
import jax.numpy as jnp
from jax.experimental import pallas as pl
from jax.experimental.pallas import tpu as pltpu

N = 16384
IN_DIM = 512
KSPLIT = IN_DIM // 2
H1 = 256
H2 = 128
OUT_DIM = 32

BLOCK = 4096
G = N // BLOCK


def _mlp_block(xa_ref, xb_ref, w1a_ref, w1b_ref, b1_ref, w2_ref, b2_ref,
               w3_ref, b3_ref, out_ref):
    h = jnp.dot(xa_ref[...].astype(jnp.bfloat16),
                w1a_ref[...].astype(jnp.bfloat16),
                preferred_element_type=jnp.float32)
    h = h + jnp.dot(xb_ref[...].astype(jnp.bfloat16),
                    w1b_ref[...].astype(jnp.bfloat16),
                    preferred_element_type=jnp.float32)
    h = jnp.maximum(h + b1_ref[...], 0.0)
    h = jnp.dot(h.astype(jnp.bfloat16), w2_ref[...].astype(jnp.bfloat16),
                preferred_element_type=jnp.float32)
    h = jnp.maximum(h + b2_ref[...], 0.0)
    h = jnp.dot(h.astype(jnp.bfloat16), w3_ref[...].astype(jnp.bfloat16),
                preferred_element_type=jnp.float32)
    out_ref[...] = h + b3_ref[...]


def kernel(x, W1, b1, W2, b2, W3, b3):
    b1r = b1.reshape(1, H1)
    b2r = b2.reshape(1, H2)
    b3r = b3.reshape(1, OUT_DIM)
    return pl.pallas_call(
        _mlp_block,
        grid=(G,),
        in_specs=[
            pl.BlockSpec((BLOCK, KSPLIT), lambda i: (i, 0)),
            pl.BlockSpec((BLOCK, KSPLIT), lambda i: (i, 1)),
            pl.BlockSpec((KSPLIT, H1), lambda i: (0, 0)),
            pl.BlockSpec((KSPLIT, H1), lambda i: (1, 0)),
            pl.BlockSpec((1, H1), lambda i: (0, 0)),
            pl.BlockSpec((H1, H2), lambda i: (0, 0)),
            pl.BlockSpec((1, H2), lambda i: (0, 0)),
            pl.BlockSpec((H2, OUT_DIM), lambda i: (0, 0)),
            pl.BlockSpec((1, OUT_DIM), lambda i: (0, 0)),
        ],
        out_specs=pl.BlockSpec((BLOCK, OUT_DIM), lambda i: (i, 0)),
        out_shape=jax.ShapeDtypeStruct((N, OUT_DIM), jnp.float32),
        compiler_params=pltpu.CompilerParams(
            dimension_semantics=("arbitrary",),
        ),
    )(x, x, W1, W1, b1r, W2, b2r, W3, b3r)
